# baseline (speedup 1.0000x reference)
import functools

import jax
import jax.numpy as jnp
from jax import lax
from jax.experimental import pallas as pl
from jax.experimental.pallas import tpu as pltpu
from jax.experimental.pallas import tpu_sc as plsc

NC = 2
NS = 16
NW = NC * NS

_EB = 80


_KG = 5


def _sc_counts(dst3, np_pad):
    nw, nit, eb = dst3.shape
    rs = np_pad // NS
    ngrp = nit // _KG
    mesh = plsc.VectorSubcoreMesh(core_axis_name="c", subcore_axis_name="s")

    @functools.partial(
        pl.kernel,
        mesh=mesh,
        out_type=jax.ShapeDtypeStruct((NC, np_pad), jnp.float32),
        scratch_types=[
            pltpu.VMEM((nit, eb), jnp.int32),
            pltpu.VMEM((eb,), jnp.float32),
            pltpu.VMEM((rs,), jnp.float32),
            pltpu.VMEM_SHARED((np_pad,), jnp.float32),
            [pltpu.SemaphoreType.DMA for _ in range(_KG)],
        ],
    )
    def k(dst_hbm, out_hbm, idx_v, ones_v, zbuf_v, acc_sh, sems):
        c = lax.axis_index("c")
        s = lax.axis_index("s")
        w = s * NC + c
        zero16 = jnp.zeros((16,), jnp.float32)
        one16 = jnp.ones((16,), jnp.float32)

        def fill_z(i, carry):
            zbuf_v[pl.ds(i * 16, 16)] = zero16
            return carry

        lax.fori_loop(0, rs // 16, fill_z, 0)

        def fill_o(i, carry):
            ones_v[pl.ds(i * 16, 16)] = one16
            return carry

        lax.fori_loop(0, eb // 16, fill_o, 0)
        pltpu.sync_copy(dst_hbm.at[w], idx_v)
        pltpu.sync_copy(zbuf_v, acc_sh.at[pl.ds(s * rs, rs)])
        plsc.subcore_barrier()

        for b in range(_KG):
            pltpu.async_copy(ones_v, acc_sh.at[idx_v.at[b]], sems[b],
                             add=True)

        def grp(i, carry):
            for b in range(_KG):
                j = i * _KG + b
                pltpu.make_async_copy(ones_v, acc_sh.at[idx_v.at[j]],
                                      sems[b]).wait()
                pltpu.async_copy(ones_v, acc_sh.at[idx_v.at[j + _KG]],
                                 sems[b], add=True)
            return carry

        lax.fori_loop(0, ngrp - 1, grp, 0)
        for b in range(_KG):
            j = (ngrp - 1) * _KG + b
            pltpu.make_async_copy(ones_v, acc_sh.at[idx_v.at[j]],
                                  sems[b]).wait()
        plsc.subcore_barrier()
        pltpu.sync_copy(acc_sh.at[pl.ds(s * rs, rs)], zbuf_v)
        pltpu.sync_copy(zbuf_v, out_hbm.at[c, pl.ds(s * rs, rs)])

    return k(dst3)


def _sc_agg(g, src3, dst3, np_pad):
    n, f = g.shape
    nw, nit, eb = src3.shape
    rs = np_pad // NS
    zb = 16
    kg = 3
    ngrp = nit // kg
    mesh = plsc.VectorSubcoreMesh(core_axis_name="c", subcore_axis_name="s")

    @functools.partial(
        pl.kernel,
        mesh=mesh,
        out_type=jax.ShapeDtypeStruct((NC, np_pad, f), jnp.float32),
        scratch_types=[
            pltpu.VMEM((nit, eb), jnp.int32),
            [pltpu.VMEM((eb,), jnp.int32) for _ in range(kg)],
            [pltpu.VMEM((eb, f), jnp.float32) for _ in range(kg)],
            pltpu.VMEM((zb, f), jnp.float32),
            pltpu.VMEM_SHARED((np_pad, f), jnp.float32),
            [pltpu.SemaphoreType.DMA for _ in range(kg)],
            [pltpu.SemaphoreType.DMA for _ in range(kg)],
            [pltpu.SemaphoreType.DMA for _ in range(kg)],
        ],
    )
    def k(g_hbm, src_hbm, dst_hbm, out_hbm, sidx_v, dring, rows, zbuf_v,
          acc_sh, gsems, dsems, ssems):
        c = lax.axis_index("c")
        s = lax.axis_index("s")
        w = s * NC + c
        zero16 = jnp.zeros((16,), jnp.float32)
        lanes = f // 16

        pltpu.sync_copy(src_hbm.at[w], sidx_v)

        def fill_z(j, carry):
            zbuf_v[j // lanes, pl.ds((j % lanes) * 16, 16)] = zero16
            return carry

        lax.fori_loop(0, zb * lanes, fill_z, 0)

        def zero_chunk(j, carry):
            pltpu.sync_copy(zbuf_v, acc_sh.at[pl.ds(s * rs + j * zb, zb)])
            return carry

        lax.fori_loop(0, rs // zb, zero_chunk, 0)
        plsc.subcore_barrier()

        def issue(j, b):
            pltpu.async_copy(dst_hbm.at[w, j], dring[b], dsems[b])
            pltpu.async_copy(g_hbm.at[sidx_v.at[j]], rows[b], gsems[b])

        def wait_gd(j, b):
            pltpu.make_async_copy(g_hbm.at[sidx_v.at[j]], rows[b],
                                  gsems[b]).wait()
            pltpu.make_async_copy(dst_hbm.at[w, j], dring[b],
                                  dsems[b]).wait()

        def scat(b):
            pltpu.async_copy(rows[b], acc_sh.at[dring[b]], ssems[b],
                             add=True)

        def wait_scat(b):
            pltpu.make_async_copy(rows[b], acc_sh.at[dring[b]],
                                  ssems[b]).wait()

        issue(0, 0)
        issue(1, 1)
        wait_gd(0, 0)
        scat(0)
        issue(2, 2)
        wait_gd(1, 1)
        scat(1)
        wait_scat(0)
        issue(3, 0)
        wait_gd(2, 2)
        scat(2)
        wait_scat(1)
        issue(4, 1)

        def grp(i, carry):
            for b in range(kg):
                j = i * kg + b
                wait_gd(j, b)
                scat(b)
                bu = (b + 2) % kg

                @pl.when(j + 2 < nit)
                def _():
                    wait_scat(bu)
                    issue(j + 2, bu)

            return carry

        lax.fori_loop(1, ngrp, grp, 0)

        for b in range(kg):
            wait_scat(b)
        plsc.subcore_barrier()

        def writeout(j, carry):
            pltpu.sync_copy(acc_sh.at[pl.ds(s * rs + j * zb, zb)], zbuf_v)
            pltpu.sync_copy(zbuf_v, out_hbm.at[c, pl.ds(s * rs + j * zb, zb)])
            return carry

        lax.fori_loop(0, rs // zb, writeout, 0)

    return k(g, src3, dst3)


def _fkan_body(xb, lng, lnb, swp_ref, bwt, bb, gvals, inv_denom):
    mu = jnp.mean(xb, axis=1, keepdims=True)
    xc = xb - mu
    var = jnp.mean(xc * xc, axis=1, keepdims=True)
    xn = xc * lax.rsqrt(var + 1e-5) * lng + lnb
    acc = jnp.dot(jax.nn.silu(xb), bwt, preferred_element_type=jnp.float32)
    for gi, gv in enumerate(gvals):
        t = (xn - gv) * inv_denom
        acc = acc + jnp.dot(jnp.exp(-t * t), swp_ref[gi],
                            preferred_element_type=jnp.float32)
    return acc + bb


def _tc_fkan0(x, cnt3, lng, lnb, swp, bwt, bb, gvals, inv_denom, bm):
    n, d = x.shape
    h = bwt.shape[1]
    nb = n // bm

    def body(x_ref, cnt_ref, lng_ref, lnb_ref, swp_ref, bwt_ref, bb_ref,
             g_ref, dinv_ref):
        deg = cnt_ref[0] + cnt_ref[1] + 1.0
        dinv = lax.rsqrt(deg)
        hval = _fkan_body(x_ref[...], lng_ref[...], lnb_ref[...], swp_ref,
                          bwt_ref[...], bb_ref[...], gvals, inv_denom)
        g_ref[...] = dinv * hval
        dinv_ref[...] = dinv

    return pl.pallas_call(
        body,
        grid=(nb,),
        in_specs=[
            pl.BlockSpec((bm, d), lambda i: (i, 0)),
            pl.BlockSpec((2, bm, 1), lambda i: (0, i, 0)),
            pl.BlockSpec((1, d), lambda i: (0, 0)),
            pl.BlockSpec((1, d), lambda i: (0, 0)),
            pl.BlockSpec(swp.shape, lambda i: (0, 0, 0)),
            pl.BlockSpec((d, h), lambda i: (0, 0)),
            pl.BlockSpec((1, h), lambda i: (0, 0)),
        ],
        out_specs=[
            pl.BlockSpec((bm, h), lambda i: (i, 0)),
            pl.BlockSpec((bm, 1), lambda i: (i, 0)),
        ],
        out_shape=[
            jax.ShapeDtypeStruct((n, h), jnp.float32),
            jax.ShapeDtypeStruct((n, 1), jnp.float32),
        ],
    )(x, cnt3, lng, lnb, swp, bwt, bb)


def _tc_post_fkan(p, g0, dinv, gbias, lng, lnb, swp, bwt, bb, gvals,
                  inv_denom, bm):
    n, h = g0.shape
    np_pad = p.shape[1]
    h2 = bwt.shape[1]
    nb = n // bm

    def body(p_ref, g0_ref, dinv_ref, gb_ref, lng_ref, lnb_ref, swp_ref,
             bwt_ref, bb_ref, g1_ref):
        dinv = dinv_ref[...]
        a = dinv * (p_ref[0] + p_ref[1] + g0_ref[...]) + gb_ref[...]
        a = jax.nn.silu(a)
        hval = _fkan_body(a, lng_ref[...], lnb_ref[...], swp_ref,
                          bwt_ref[...], bb_ref[...], gvals, inv_denom)
        g1_ref[...] = dinv * hval

    return pl.pallas_call(
        body,
        grid=(nb,),
        in_specs=[
            pl.BlockSpec((2, bm, h), lambda i: (0, i, 0)),
            pl.BlockSpec((bm, h), lambda i: (i, 0)),
            pl.BlockSpec((bm, 1), lambda i: (i, 0)),
            pl.BlockSpec((1, h), lambda i: (0, 0)),
            pl.BlockSpec((1, h), lambda i: (0, 0)),
            pl.BlockSpec((1, h), lambda i: (0, 0)),
            pl.BlockSpec(swp.shape, lambda i: (0, 0, 0)),
            pl.BlockSpec((h, h2), lambda i: (0, 0)),
            pl.BlockSpec((1, h2), lambda i: (0, 0)),
        ],
        out_specs=pl.BlockSpec((bm, h2), lambda i: (i, 0)),
        out_shape=jax.ShapeDtypeStruct((n, h2), jnp.float32),
    )(p, g0, dinv, gbias, lng, lnb, swp, bwt, bb)


def _tc_post_pool(p, g1, dinv, gbias, bm):
    n, h = g1.shape
    nb = n // bm
    inv_n = 1.0 / n

    def body(p_ref, g1_ref, dinv_ref, gb_ref, out_ref):
        i = pl.program_id(0)
        a = dinv_ref[...] * (p_ref[0] + p_ref[1] + g1_ref[...]) + gb_ref[...]
        z = jax.nn.silu(a)
        colsum = jnp.sum(z, axis=0, keepdims=True)

        @pl.when(i == 0)
        def _():
            out_ref[...] = jnp.zeros_like(out_ref)

        out_ref[...] += colsum

        @pl.when(i == nb - 1)
        def _():
            out_ref[...] *= inv_n

    return pl.pallas_call(
        body,
        grid=(nb,),
        in_specs=[
            pl.BlockSpec((2, bm, h), lambda i: (0, i, 0)),
            pl.BlockSpec((bm, h), lambda i: (i, 0)),
            pl.BlockSpec((bm, 1), lambda i: (i, 0)),
            pl.BlockSpec((1, h), lambda i: (0, 0)),
        ],
        out_specs=pl.BlockSpec((1, h), lambda i: (0, 0)),
        out_shape=jax.ShapeDtypeStruct((1, h), jnp.float32),
    )(p, g1, dinv, gbias)


def _tc_head(pooled, lng, lnb, swp, bwt, bb, c_real, gvals, inv_denom):
    h = pooled.shape[1]
    cp = bwt.shape[1]

    def body(x_ref, lng_ref, lnb_ref, swp_ref, bwt_ref, bb_ref, out_ref):
        y = _fkan_body(x_ref[...], lng_ref[...], lnb_ref[...], swp_ref,
                       bwt_ref[...], bb_ref[...], gvals, inv_denom)
        col = lax.broadcasted_iota(jnp.int32, (1, cp), 1)
        mask = col < c_real
        ymask = jnp.where(mask, y, -1e30)
        m = jnp.max(ymask, axis=1, keepdims=True)
        ez = jnp.where(mask, jnp.exp(y - m), 0.0)
        lse = jnp.log(jnp.sum(ez, axis=1, keepdims=True)) + m
        out_ref[...] = y - lse

    return pl.pallas_call(
        body,
        in_specs=[
            pl.BlockSpec((1, h), lambda: (0, 0)),
            pl.BlockSpec((1, h), lambda: (0, 0)),
            pl.BlockSpec((1, h), lambda: (0, 0)),
            pl.BlockSpec(swp.shape, lambda: (0, 0, 0)),
            pl.BlockSpec((h, cp), lambda: (0, 0)),
            pl.BlockSpec((1, cp), lambda: (0, 0)),
        ],
        out_specs=pl.BlockSpec((1, cp), lambda: (0, 0)),
        out_shape=jax.ShapeDtypeStruct((1, cp), jnp.float32),
    )(pooled, lng, lnb, swp, bwt, bb)


def _perm_spline(sw, d_in, g):
    return jnp.transpose(jnp.reshape(jnp.transpose(sw), (d_in, g, -1)),
                         (1, 0, 2))


def kernel(x, edge_index, batch, ln_g0, ln_b0, sw0, bw0, bb0, gb0,
           ln_g1, ln_b1, sw1, bw1, bb1, gb1, ln_gr, ln_br, swr, bwr, bbr):
    n, d = x.shape
    h = bw0.shape[0]
    c = swr.shape[0]
    g = sw0.shape[1] // d
    gvals = [-2.0 + 4.0 * i / (g - 1) for i in range(g)]
    inv_denom = (g - 1) / 4.0
    np_pad = ((n + NS * 64 - 1) // (NS * 64)) * (NS * 64)
    bm = 1000

    dst_c = jnp.reshape(edge_index[1], (NW, -1, _EB))
    eba = 64
    ew = edge_index.shape[1] // NW
    ewp = ((ew + 3 * eba - 1) // (3 * eba)) * (3 * eba)
    src2 = jnp.reshape(edge_index[0], (NW, ew))
    dst2 = jnp.reshape(edge_index[1], (NW, ew))
    pad_s = jnp.zeros((NW, ewp - ew), jnp.int32)
    pad_d = jnp.full((NW, ewp - ew), np_pad - 1, jnp.int32)
    src = jnp.reshape(jnp.concatenate([src2, pad_s], axis=1), (NW, -1, eba))
    dst = jnp.reshape(jnp.concatenate([dst2, pad_d], axis=1), (NW, -1, eba))

    row = lambda v: jnp.reshape(v, (1, -1))
    swp0 = _perm_spline(sw0, d, g)
    swp1 = _perm_spline(sw1, h, g)
    cp = 128
    swpr = jnp.zeros((g, h, cp), jnp.float32).at[:, :, :c].set(
        _perm_spline(swr, h, g))
    bwtr = jnp.zeros((h, cp), jnp.float32).at[:, :c].set(jnp.transpose(bwr))
    bbr_p = jnp.zeros((1, cp), jnp.float32).at[0, :c].set(bbr)

    cnt = _sc_counts(dst_c, np_pad)
    cnt3 = jnp.reshape(cnt, (NC, np_pad, 1))

    g0, dinv = _tc_fkan0(x, cnt3, row(ln_g0), row(ln_b0), swp0,
                         jnp.transpose(bw0), row(bb0), gvals, inv_denom, bm)
    p0 = _sc_agg(g0, src, dst, np_pad)
    g1 = _tc_post_fkan(p0, g0, dinv, row(gb0), row(ln_g1), row(ln_b1), swp1,
                       jnp.transpose(bw1), row(bb1), gvals, inv_denom, bm)
    p1 = _sc_agg(g1, src, dst, np_pad)
    pooled = _tc_post_pool(p1, g1, dinv, row(gb1), bm)
    out = _tc_head(pooled, row(ln_gr), row(ln_br), swpr, bwtr, bbr_p, c,
                   gvals, inv_denom)
    return out[:, :c]


# speedup vs baseline: 1.4420x; 1.4420x over previous
import functools

import jax
import jax.numpy as jnp
from jax import lax
from jax.experimental import pallas as pl
from jax.experimental.pallas import tpu as pltpu
from jax.experimental.pallas import tpu_sc as plsc

NC = 2
NS = 16
NW = NC * NS

_EB = 80


_KG = 5


def _sc_counts(dst3, np_pad):
    nw, nit, eb = dst3.shape
    rs = np_pad // NS
    ngrp = nit // _KG
    mesh = plsc.VectorSubcoreMesh(core_axis_name="c", subcore_axis_name="s")

    @functools.partial(
        pl.kernel,
        mesh=mesh,
        out_type=jax.ShapeDtypeStruct((NC, np_pad), jnp.float32),
        scratch_types=[
            pltpu.VMEM((nit, eb), jnp.int32),
            pltpu.VMEM((eb,), jnp.float32),
            pltpu.VMEM((rs,), jnp.float32),
            pltpu.VMEM_SHARED((np_pad,), jnp.float32),
            [pltpu.SemaphoreType.DMA for _ in range(_KG)],
        ],
    )
    def k(dst_hbm, out_hbm, idx_v, ones_v, zbuf_v, acc_sh, sems):
        c = lax.axis_index("c")
        s = lax.axis_index("s")
        w = s * NC + c
        zero16 = jnp.zeros((16,), jnp.float32)
        one16 = jnp.ones((16,), jnp.float32)

        def fill_z(i, carry):
            zbuf_v[pl.ds(i * 16, 16)] = zero16
            return carry

        lax.fori_loop(0, rs // 16, fill_z, 0)

        def fill_o(i, carry):
            ones_v[pl.ds(i * 16, 16)] = one16
            return carry

        lax.fori_loop(0, eb // 16, fill_o, 0)
        pltpu.sync_copy(dst_hbm.at[w], idx_v)
        pltpu.sync_copy(zbuf_v, acc_sh.at[pl.ds(s * rs, rs)])
        plsc.subcore_barrier()

        for b in range(_KG):
            pltpu.async_copy(ones_v, acc_sh.at[idx_v.at[b]], sems[b],
                             add=True)

        def grp(i, carry):
            for b in range(_KG):
                j = i * _KG + b
                pltpu.make_async_copy(ones_v, acc_sh.at[idx_v.at[j]],
                                      sems[b]).wait()
                pltpu.async_copy(ones_v, acc_sh.at[idx_v.at[j + _KG]],
                                 sems[b], add=True)
            return carry

        lax.fori_loop(0, ngrp - 1, grp, 0)
        for b in range(_KG):
            j = (ngrp - 1) * _KG + b
            pltpu.make_async_copy(ones_v, acc_sh.at[idx_v.at[j]],
                                  sems[b]).wait()
        plsc.subcore_barrier()
        pltpu.sync_copy(acc_sh.at[pl.ds(s * rs, rs)], zbuf_v)
        pltpu.sync_copy(zbuf_v, out_hbm.at[c, pl.ds(s * rs, rs)])

    return k(dst3)


def _sc_agg(g, src3, dst3, np_pad):
    n, f = g.shape
    nw, nit, eb = src3.shape
    rs = np_pad // NS
    zb = 32
    mesh = plsc.VectorSubcoreMesh(core_axis_name="c", subcore_axis_name="s")

    @functools.partial(
        pl.kernel,
        mesh=mesh,
        out_type=jax.ShapeDtypeStruct((NC, np_pad, f), jnp.float32),
        scratch_types=[
            pltpu.VMEM((nit, eb), jnp.int32),
            pltpu.VMEM((nit, eb), jnp.int32),
            pltpu.VMEM((eb, f), jnp.float32),
            pltpu.VMEM((zb, f), jnp.float32),
            pltpu.VMEM_SHARED((np_pad, f), jnp.float32),
            pltpu.SemaphoreType.DMA,
        ],
    )
    def k(g_hbm, src_hbm, dst_hbm, out_hbm, sidx_v, didx_v, rows_v, zbuf_v,
          acc_sh, sem):
        c = lax.axis_index("c")
        s = lax.axis_index("s")
        w = s * NC + c
        zero16 = jnp.zeros((16,), jnp.float32)
        lanes = f // 16

        pltpu.sync_copy(src_hbm.at[w], sidx_v)
        pltpu.sync_copy(dst_hbm.at[w], didx_v)

        def fill_z(j, carry):
            zbuf_v[j // lanes, pl.ds((j % lanes) * 16, 16)] = zero16
            return carry

        lax.fori_loop(0, zb * lanes, fill_z, 0)

        def zero_chunk(j, carry):
            pltpu.sync_copy(zbuf_v, acc_sh.at[pl.ds(s * rs + j * zb, zb)])
            return carry

        lax.fori_loop(0, rs // zb, zero_chunk, 0)
        plsc.subcore_barrier()

        def body(j, carry):
            pltpu.async_copy(g_hbm.at[sidx_v.at[j]], rows_v, sem)
            pltpu.make_async_copy(g_hbm.at[sidx_v.at[j]], rows_v, sem).wait()
            pltpu.sync_copy(rows_v, acc_sh.at[didx_v.at[j]], add=True)
            return carry

        lax.fori_loop(0, nit, body, 0)
        plsc.subcore_barrier()

        def writeout(j, carry):
            pltpu.sync_copy(acc_sh.at[pl.ds(s * rs + j * zb, zb)], zbuf_v)
            pltpu.sync_copy(zbuf_v, out_hbm.at[c, pl.ds(s * rs + j * zb, zb)])
            return carry

        lax.fori_loop(0, rs // zb, writeout, 0)

    return k(g, src3, dst3)


def _fkan_body(xb, lng, lnb, swp_ref, bwt, bb, gvals, inv_denom):
    mu = jnp.mean(xb, axis=1, keepdims=True)
    xc = xb - mu
    var = jnp.mean(xc * xc, axis=1, keepdims=True)
    xn = xc * lax.rsqrt(var + 1e-5) * lng + lnb
    acc = jnp.dot(jax.nn.silu(xb), bwt, preferred_element_type=jnp.float32)
    for gi, gv in enumerate(gvals):
        t = (xn - gv) * inv_denom
        acc = acc + jnp.dot(jnp.exp(-t * t), swp_ref[gi],
                            preferred_element_type=jnp.float32)
    return acc + bb


def _tc_fkan0(x, cnt3, lng, lnb, swp, bwt, bb, gvals, inv_denom, bm):
    n, d = x.shape
    h = bwt.shape[1]
    nb = n // bm

    def body(x_ref, cnt_ref, lng_ref, lnb_ref, swp_ref, bwt_ref, bb_ref,
             g_ref, dinv_ref):
        deg = cnt_ref[0] + cnt_ref[1] + 1.0
        dinv = lax.rsqrt(deg)
        hval = _fkan_body(x_ref[...], lng_ref[...], lnb_ref[...], swp_ref,
                          bwt_ref[...], bb_ref[...], gvals, inv_denom)
        g_ref[...] = dinv * hval
        dinv_ref[...] = dinv

    return pl.pallas_call(
        body,
        grid=(nb,),
        in_specs=[
            pl.BlockSpec((bm, d), lambda i: (i, 0)),
            pl.BlockSpec((2, bm, 1), lambda i: (0, i, 0)),
            pl.BlockSpec((1, d), lambda i: (0, 0)),
            pl.BlockSpec((1, d), lambda i: (0, 0)),
            pl.BlockSpec(swp.shape, lambda i: (0, 0, 0)),
            pl.BlockSpec((d, h), lambda i: (0, 0)),
            pl.BlockSpec((1, h), lambda i: (0, 0)),
        ],
        out_specs=[
            pl.BlockSpec((bm, h), lambda i: (i, 0)),
            pl.BlockSpec((bm, 1), lambda i: (i, 0)),
        ],
        out_shape=[
            jax.ShapeDtypeStruct((n, h), jnp.float32),
            jax.ShapeDtypeStruct((n, 1), jnp.float32),
        ],
    )(x, cnt3, lng, lnb, swp, bwt, bb)


def _tc_post_fkan(p, g0, dinv, gbias, lng, lnb, swp, bwt, bb, gvals,
                  inv_denom, bm):
    n, h = g0.shape
    np_pad = p.shape[1]
    h2 = bwt.shape[1]
    nb = n // bm

    def body(p_ref, g0_ref, dinv_ref, gb_ref, lng_ref, lnb_ref, swp_ref,
             bwt_ref, bb_ref, g1_ref):
        dinv = dinv_ref[...]
        a = dinv * (p_ref[0] + p_ref[1] + g0_ref[...]) + gb_ref[...]
        a = jax.nn.silu(a)
        hval = _fkan_body(a, lng_ref[...], lnb_ref[...], swp_ref,
                          bwt_ref[...], bb_ref[...], gvals, inv_denom)
        g1_ref[...] = dinv * hval

    return pl.pallas_call(
        body,
        grid=(nb,),
        in_specs=[
            pl.BlockSpec((2, bm, h), lambda i: (0, i, 0)),
            pl.BlockSpec((bm, h), lambda i: (i, 0)),
            pl.BlockSpec((bm, 1), lambda i: (i, 0)),
            pl.BlockSpec((1, h), lambda i: (0, 0)),
            pl.BlockSpec((1, h), lambda i: (0, 0)),
            pl.BlockSpec((1, h), lambda i: (0, 0)),
            pl.BlockSpec(swp.shape, lambda i: (0, 0, 0)),
            pl.BlockSpec((h, h2), lambda i: (0, 0)),
            pl.BlockSpec((1, h2), lambda i: (0, 0)),
        ],
        out_specs=pl.BlockSpec((bm, h2), lambda i: (i, 0)),
        out_shape=jax.ShapeDtypeStruct((n, h2), jnp.float32),
    )(p, g0, dinv, gbias, lng, lnb, swp, bwt, bb)


def _tc_post_pool(p, g1, dinv, gbias, bm):
    n, h = g1.shape
    nb = n // bm
    inv_n = 1.0 / n

    def body(p_ref, g1_ref, dinv_ref, gb_ref, out_ref):
        i = pl.program_id(0)
        a = dinv_ref[...] * (p_ref[0] + p_ref[1] + g1_ref[...]) + gb_ref[...]
        z = jax.nn.silu(a)
        colsum = jnp.sum(z, axis=0, keepdims=True)

        @pl.when(i == 0)
        def _():
            out_ref[...] = jnp.zeros_like(out_ref)

        out_ref[...] += colsum

        @pl.when(i == nb - 1)
        def _():
            out_ref[...] *= inv_n

    return pl.pallas_call(
        body,
        grid=(nb,),
        in_specs=[
            pl.BlockSpec((2, bm, h), lambda i: (0, i, 0)),
            pl.BlockSpec((bm, h), lambda i: (i, 0)),
            pl.BlockSpec((bm, 1), lambda i: (i, 0)),
            pl.BlockSpec((1, h), lambda i: (0, 0)),
        ],
        out_specs=pl.BlockSpec((1, h), lambda i: (0, 0)),
        out_shape=jax.ShapeDtypeStruct((1, h), jnp.float32),
    )(p, g1, dinv, gbias)


def _tc_head(pooled, lng, lnb, swp, bwt, bb, c_real, gvals, inv_denom):
    h = pooled.shape[1]
    cp = bwt.shape[1]

    def body(x_ref, lng_ref, lnb_ref, swp_ref, bwt_ref, bb_ref, out_ref):
        y = _fkan_body(x_ref[...], lng_ref[...], lnb_ref[...], swp_ref,
                       bwt_ref[...], bb_ref[...], gvals, inv_denom)
        col = lax.broadcasted_iota(jnp.int32, (1, cp), 1)
        mask = col < c_real
        ymask = jnp.where(mask, y, -1e30)
        m = jnp.max(ymask, axis=1, keepdims=True)
        ez = jnp.where(mask, jnp.exp(y - m), 0.0)
        lse = jnp.log(jnp.sum(ez, axis=1, keepdims=True)) + m
        out_ref[...] = y - lse

    return pl.pallas_call(
        body,
        in_specs=[
            pl.BlockSpec((1, h), lambda: (0, 0)),
            pl.BlockSpec((1, h), lambda: (0, 0)),
            pl.BlockSpec((1, h), lambda: (0, 0)),
            pl.BlockSpec(swp.shape, lambda: (0, 0, 0)),
            pl.BlockSpec((h, cp), lambda: (0, 0)),
            pl.BlockSpec((1, cp), lambda: (0, 0)),
        ],
        out_specs=pl.BlockSpec((1, cp), lambda: (0, 0)),
        out_shape=jax.ShapeDtypeStruct((1, cp), jnp.float32),
    )(pooled, lng, lnb, swp, bwt, bb)


def _perm_spline(sw, d_in, g):
    return jnp.transpose(jnp.reshape(jnp.transpose(sw), (d_in, g, -1)),
                         (1, 0, 2))


def kernel(x, edge_index, batch, ln_g0, ln_b0, sw0, bw0, bb0, gb0,
           ln_g1, ln_b1, sw1, bw1, bb1, gb1, ln_gr, ln_br, swr, bwr, bbr):
    n, d = x.shape
    h = bw0.shape[0]
    c = swr.shape[0]
    g = sw0.shape[1] // d
    gvals = [-2.0 + 4.0 * i / (g - 1) for i in range(g)]
    inv_denom = (g - 1) / 4.0
    np_pad = ((n + NS * 64 - 1) // (NS * 64)) * (NS * 64)
    bm = 1000

    src = jnp.reshape(edge_index[0], (NW, -1, _EB))
    dst = jnp.reshape(edge_index[1], (NW, -1, _EB))
    dst_c = dst

    row = lambda v: jnp.reshape(v, (1, -1))
    swp0 = _perm_spline(sw0, d, g)
    swp1 = _perm_spline(sw1, h, g)
    cp = 128
    swpr = jnp.zeros((g, h, cp), jnp.float32).at[:, :, :c].set(
        _perm_spline(swr, h, g))
    bwtr = jnp.zeros((h, cp), jnp.float32).at[:, :c].set(jnp.transpose(bwr))
    bbr_p = jnp.zeros((1, cp), jnp.float32).at[0, :c].set(bbr)

    cnt = _sc_counts(dst_c, np_pad)
    cnt3 = jnp.reshape(cnt, (NC, np_pad, 1))

    g0, dinv = _tc_fkan0(x, cnt3, row(ln_g0), row(ln_b0), swp0,
                         jnp.transpose(bw0), row(bb0), gvals, inv_denom, bm)
    p0 = _sc_agg(g0, src, dst, np_pad)
    g1 = _tc_post_fkan(p0, g0, dinv, row(gb0), row(ln_g1), row(ln_b1), swp1,
                       jnp.transpose(bw1), row(bb1), gvals, inv_denom, bm)
    p1 = _sc_agg(g1, src, dst, np_pad)
    pooled = _tc_post_pool(p1, g1, dinv, row(gb1), bm)
    out = _tc_head(pooled, row(ln_gr), row(ln_br), swpr, bwtr, bbr_p, c,
                   gvals, inv_denom)
    return out[:, :c]


# speedup vs baseline: 1.6399x; 1.1372x over previous
import functools

import jax
import jax.numpy as jnp
from jax import lax
from jax.experimental import pallas as pl
from jax.experimental.pallas import tpu as pltpu
from jax.experimental.pallas import tpu_sc as plsc

NC = 2
NS = 16
NW = NC * NS

_EB = 80


_KG = 5


def _sc_counts(dst3, np_pad):
    nw, nit, eb = dst3.shape
    rs = np_pad // NS
    ngrp = nit // _KG
    mesh = plsc.VectorSubcoreMesh(core_axis_name="c", subcore_axis_name="s")

    @functools.partial(
        pl.kernel,
        mesh=mesh,
        out_type=jax.ShapeDtypeStruct((NC, np_pad), jnp.float32),
        scratch_types=[
            pltpu.VMEM((nit, eb), jnp.int32),
            pltpu.VMEM((eb,), jnp.float32),
            pltpu.VMEM((rs,), jnp.float32),
            pltpu.VMEM_SHARED((np_pad,), jnp.float32),
            [pltpu.SemaphoreType.DMA for _ in range(_KG)],
        ],
    )
    def k(dst_hbm, out_hbm, idx_v, ones_v, zbuf_v, acc_sh, sems):
        c = lax.axis_index("c")
        s = lax.axis_index("s")
        w = s * NC + c
        zero16 = jnp.zeros((16,), jnp.float32)
        one16 = jnp.ones((16,), jnp.float32)

        def fill_z(i, carry):
            zbuf_v[pl.ds(i * 16, 16)] = zero16
            return carry

        lax.fori_loop(0, rs // 16, fill_z, 0)

        def fill_o(i, carry):
            ones_v[pl.ds(i * 16, 16)] = one16
            return carry

        lax.fori_loop(0, eb // 16, fill_o, 0)
        pltpu.sync_copy(dst_hbm.at[w], idx_v)
        pltpu.sync_copy(zbuf_v, acc_sh.at[pl.ds(s * rs, rs)])
        plsc.subcore_barrier()

        for b in range(_KG):
            pltpu.async_copy(ones_v, acc_sh.at[idx_v.at[b]], sems[b],
                             add=True)

        def grp(i, carry):
            for b in range(_KG):
                j = i * _KG + b
                pltpu.make_async_copy(ones_v, acc_sh.at[idx_v.at[j]],
                                      sems[b]).wait()
                pltpu.async_copy(ones_v, acc_sh.at[idx_v.at[j + _KG]],
                                 sems[b], add=True)
            return carry

        lax.fori_loop(0, ngrp - 1, grp, 0)
        for b in range(_KG):
            j = (ngrp - 1) * _KG + b
            pltpu.make_async_copy(ones_v, acc_sh.at[idx_v.at[j]],
                                  sems[b]).wait()
        plsc.subcore_barrier()
        pltpu.sync_copy(acc_sh.at[pl.ds(s * rs, rs)], zbuf_v)
        pltpu.sync_copy(zbuf_v, out_hbm.at[c, pl.ds(s * rs, rs)])

    return k(dst3)


def _sc_agg(g, src3, dst3, np_pad):
    n, f = g.shape
    nw, nit, eb = src3.shape
    rs = np_pad // NS
    zb = 32
    mesh = plsc.VectorSubcoreMesh(core_axis_name="c", subcore_axis_name="s")

    @functools.partial(
        pl.kernel,
        mesh=mesh,
        out_type=jax.ShapeDtypeStruct((NC, np_pad, f), jnp.float32),
        scratch_types=[
            pltpu.VMEM((nit, eb), jnp.int32),
            pltpu.VMEM((nit, eb), jnp.int32),
            pltpu.VMEM((eb, f), jnp.float32),
            pltpu.VMEM((zb, f), jnp.float32),
            pltpu.VMEM_SHARED((np_pad, f), jnp.float32),
            pltpu.SemaphoreType.DMA,
        ],
    )
    def k(g_hbm, src_hbm, dst_hbm, out_hbm, sidx_v, didx_v, rows_v, zbuf_v,
          acc_sh, sem):
        c = lax.axis_index("c")
        s = lax.axis_index("s")
        w = s * NC + c
        zero16 = jnp.zeros((16,), jnp.float32)
        lanes = f // 16

        pltpu.sync_copy(src_hbm.at[w], sidx_v)
        pltpu.sync_copy(dst_hbm.at[w], didx_v)

        def fill_z(j, carry):
            zbuf_v[j // lanes, pl.ds((j % lanes) * 16, 16)] = zero16
            return carry

        lax.fori_loop(0, zb * lanes, fill_z, 0)

        def zero_chunk(j, carry):
            pltpu.sync_copy(zbuf_v, acc_sh.at[pl.ds(s * rs + j * zb, zb)])
            return carry

        lax.fori_loop(0, rs // zb, zero_chunk, 0)
        plsc.subcore_barrier()

        def body(j, carry):
            pltpu.async_copy(g_hbm.at[sidx_v.at[j]], rows_v, sem)
            pltpu.make_async_copy(g_hbm.at[sidx_v.at[j]], rows_v, sem).wait()
            pltpu.sync_copy(rows_v, acc_sh.at[didx_v.at[j]], add=True)
            return carry

        lax.fori_loop(0, nit, body, 0)
        plsc.subcore_barrier()

        def writeout(j, carry):
            pltpu.sync_copy(acc_sh.at[pl.ds(s * rs + j * zb, zb)], zbuf_v)
            pltpu.sync_copy(zbuf_v, out_hbm.at[c, pl.ds(s * rs + j * zb, zb)])
            return carry

        lax.fori_loop(0, rs // zb, writeout, 0)

    return k(g, src3, dst3)


def _fkan_body(xb, lng, lnb, swp_ref, bwt, bb, gvals, inv_denom):
    mu = jnp.mean(xb, axis=1, keepdims=True)
    xc = xb - mu
    var = jnp.mean(xc * xc, axis=1, keepdims=True)
    xn = xc * lax.rsqrt(var + 1e-5) * lng + lnb
    acc = jnp.dot(jax.nn.silu(xb), bwt, preferred_element_type=jnp.float32)
    for gi, gv in enumerate(gvals):
        t = (xn - gv) * inv_denom
        acc = acc + jnp.dot(jnp.exp(-t * t), swp_ref[gi],
                            preferred_element_type=jnp.float32)
    return acc + bb


def _tc_fkan0(x, cnt3, lng, lnb, swp, bwt, bb, gvals, inv_denom, bm):
    n, d = x.shape
    h = bwt.shape[1]
    nb = n // bm

    def body(x_ref, cnt_ref, lng_ref, lnb_ref, swp_ref, bwt_ref, bb_ref,
             g_ref, dinv_ref):
        deg = cnt_ref[0] + cnt_ref[1] + 1.0
        dinv = lax.rsqrt(deg)
        hval = _fkan_body(x_ref[...], lng_ref[...], lnb_ref[...], swp_ref,
                          bwt_ref[...], bb_ref[...], gvals, inv_denom)
        g_ref[...] = dinv * hval
        dinv_ref[...] = dinv

    return pl.pallas_call(
        body,
        grid=(nb,),
        in_specs=[
            pl.BlockSpec((bm, d), lambda i: (i, 0)),
            pl.BlockSpec((2, bm, 1), lambda i: (0, i, 0)),
            pl.BlockSpec((1, d), lambda i: (0, 0)),
            pl.BlockSpec((1, d), lambda i: (0, 0)),
            pl.BlockSpec(swp.shape, lambda i: (0, 0, 0)),
            pl.BlockSpec((d, h), lambda i: (0, 0)),
            pl.BlockSpec((1, h), lambda i: (0, 0)),
        ],
        out_specs=[
            pl.BlockSpec((bm, h), lambda i: (i, 0)),
            pl.BlockSpec((bm, 1), lambda i: (i, 0)),
        ],
        out_shape=[
            jax.ShapeDtypeStruct((n, h), jnp.float32),
            jax.ShapeDtypeStruct((n, 1), jnp.float32),
        ],
    )(x, cnt3, lng, lnb, swp, bwt, bb)


def _tc_post_fkan(p, g0, dinv, gbias, lng, lnb, swp, bwt, bb, gvals,
                  inv_denom, bm):
    n, h = g0.shape
    np_pad = p.shape[1]
    h2 = bwt.shape[1]
    nb = n // bm

    def body(p_ref, g0_ref, dinv_ref, gb_ref, lng_ref, lnb_ref, swp_ref,
             bwt_ref, bb_ref, g1_ref):
        dinv = dinv_ref[...]
        a = dinv * (p_ref[0] + p_ref[1] + g0_ref[...]) + gb_ref[...]
        a = jax.nn.silu(a)
        hval = _fkan_body(a, lng_ref[...], lnb_ref[...], swp_ref,
                          bwt_ref[...], bb_ref[...], gvals, inv_denom)
        g1_ref[...] = dinv * hval

    return pl.pallas_call(
        body,
        grid=(nb,),
        in_specs=[
            pl.BlockSpec((2, bm, h), lambda i: (0, i, 0)),
            pl.BlockSpec((bm, h), lambda i: (i, 0)),
            pl.BlockSpec((bm, 1), lambda i: (i, 0)),
            pl.BlockSpec((1, h), lambda i: (0, 0)),
            pl.BlockSpec((1, h), lambda i: (0, 0)),
            pl.BlockSpec((1, h), lambda i: (0, 0)),
            pl.BlockSpec(swp.shape, lambda i: (0, 0, 0)),
            pl.BlockSpec((h, h2), lambda i: (0, 0)),
            pl.BlockSpec((1, h2), lambda i: (0, 0)),
        ],
        out_specs=pl.BlockSpec((bm, h2), lambda i: (i, 0)),
        out_shape=jax.ShapeDtypeStruct((n, h2), jnp.float32),
    )(p, g0, dinv, gbias, lng, lnb, swp, bwt, bb)


def _tc_post_pool(p, g1, dinv, gbias, bm):
    n, h = g1.shape
    nb = n // bm
    inv_n = 1.0 / n

    def body(p_ref, g1_ref, dinv_ref, gb_ref, out_ref):
        i = pl.program_id(0)
        a = dinv_ref[...] * (p_ref[0] + p_ref[1] + g1_ref[...]) + gb_ref[...]
        z = jax.nn.silu(a)
        colsum = jnp.sum(z, axis=0, keepdims=True)

        @pl.when(i == 0)
        def _():
            out_ref[...] = jnp.zeros_like(out_ref)

        out_ref[...] += colsum

        @pl.when(i == nb - 1)
        def _():
            out_ref[...] *= inv_n

    return pl.pallas_call(
        body,
        grid=(nb,),
        in_specs=[
            pl.BlockSpec((2, bm, h), lambda i: (0, i, 0)),
            pl.BlockSpec((bm, h), lambda i: (i, 0)),
            pl.BlockSpec((bm, 1), lambda i: (i, 0)),
            pl.BlockSpec((1, h), lambda i: (0, 0)),
        ],
        out_specs=pl.BlockSpec((1, h), lambda i: (0, 0)),
        out_shape=jax.ShapeDtypeStruct((1, h), jnp.float32),
    )(p, g1, dinv, gbias)


def _tc_head(pooled, lng, lnb, swp, bwt, bb, c_real, gvals, inv_denom):
    h = pooled.shape[1]
    cp = bwt.shape[1]

    def body(x_ref, lng_ref, lnb_ref, swp_ref, bwt_ref, bb_ref, out_ref):
        y = _fkan_body(x_ref[...], lng_ref[...], lnb_ref[...], swp_ref,
                       bwt_ref[...], bb_ref[...], gvals, inv_denom)
        col = lax.broadcasted_iota(jnp.int32, (1, cp), 1)
        mask = col < c_real
        ymask = jnp.where(mask, y, -1e30)
        m = jnp.max(ymask, axis=1, keepdims=True)
        ez = jnp.where(mask, jnp.exp(y - m), 0.0)
        lse = jnp.log(jnp.sum(ez, axis=1, keepdims=True)) + m
        out_ref[...] = y - lse

    return pl.pallas_call(
        body,
        in_specs=[
            pl.BlockSpec((1, h), lambda: (0, 0)),
            pl.BlockSpec((1, h), lambda: (0, 0)),
            pl.BlockSpec((1, h), lambda: (0, 0)),
            pl.BlockSpec(swp.shape, lambda: (0, 0, 0)),
            pl.BlockSpec((h, cp), lambda: (0, 0)),
            pl.BlockSpec((1, cp), lambda: (0, 0)),
        ],
        out_specs=pl.BlockSpec((1, cp), lambda: (0, 0)),
        out_shape=jax.ShapeDtypeStruct((1, cp), jnp.float32),
    )(pooled, lng, lnb, swp, bwt, bb)


def _perm_spline(sw, d_in, g):
    return jnp.transpose(jnp.reshape(jnp.transpose(sw), (d_in, g, -1)),
                         (1, 0, 2))


def kernel(x, edge_index, batch, ln_g0, ln_b0, sw0, bw0, bb0, gb0,
           ln_g1, ln_b1, sw1, bw1, bb1, gb1, ln_gr, ln_br, swr, bwr, bbr):
    n, d = x.shape
    h = bw0.shape[0]
    c = swr.shape[0]
    g = sw0.shape[1] // d
    gvals = [-2.0 + 4.0 * i / (g - 1) for i in range(g)]
    inv_denom = (g - 1) / 4.0
    np_pad = ((n + NS * 64 - 1) // (NS * 64)) * (NS * 64)
    bm = 1000

    dst_c = jnp.reshape(edge_index[1], (NW, -1, _EB))
    eba = 125
    src = jnp.reshape(edge_index[0], (NW, -1, eba))
    dst = jnp.reshape(edge_index[1], (NW, -1, eba))

    row = lambda v: jnp.reshape(v, (1, -1))
    swp0 = _perm_spline(sw0, d, g)
    swp1 = _perm_spline(sw1, h, g)
    cp = 128
    swpr = jnp.zeros((g, h, cp), jnp.float32).at[:, :, :c].set(
        _perm_spline(swr, h, g))
    bwtr = jnp.zeros((h, cp), jnp.float32).at[:, :c].set(jnp.transpose(bwr))
    bbr_p = jnp.zeros((1, cp), jnp.float32).at[0, :c].set(bbr)

    cnt = _sc_counts(dst_c, np_pad)
    cnt3 = jnp.reshape(cnt, (NC, np_pad, 1))

    g0, dinv = _tc_fkan0(x, cnt3, row(ln_g0), row(ln_b0), swp0,
                         jnp.transpose(bw0), row(bb0), gvals, inv_denom, bm)
    p0 = _sc_agg(g0, src, dst, np_pad)
    g1 = _tc_post_fkan(p0, g0, dinv, row(gb0), row(ln_g1), row(ln_b1), swp1,
                       jnp.transpose(bw1), row(bb1), gvals, inv_denom, bm)
    p1 = _sc_agg(g1, src, dst, np_pad)
    pooled = _tc_post_pool(p1, g1, dinv, row(gb1), bm)
    out = _tc_head(pooled, row(ln_gr), row(ln_br), swpr, bwtr, bbr_p, c,
                   gvals, inv_denom)
    return out[:, :c]
